# baseline (speedup 1.0000x reference)
import jax
import jax.numpy as jnp
from jax import lax
from jax.experimental import pallas as pl
from jax.experimental.pallas import tpu as pltpu
from jax.experimental.pallas import tpu_sc as plsc

_NC, _NS = 2, 16
_NW = _NC * _NS
_CHUNK = 400
_NBUF = 4


def _embed_body(tokens_hbm, oidx_hbm, table_hbm, out_hbm,
                idx_v, oidx_v, rows_v, gsem, isem, osem, ssem):
    n = tokens_hbm.shape[0]
    D = table_hbm.shape[1]
    n_per_w = n // _NW
    num_chunks = n_per_w // _CHUNK
    row_bytes = _CHUNK * D * 4
    wid = lax.axis_index("s") * _NC + lax.axis_index("c")
    base = wid * n_per_w

    def tok_sl(g):
        return tokens_hbm.at[pl.ds(base + g * _CHUNK, _CHUNK)]

    def oix_sl(g):
        return oidx_hbm.at[pl.ds(base + g * _CHUNK, _CHUNK)]

    def gather_start(b):
        pltpu.async_copy(table_hbm.at[idx_v.at[b]], rows_v.at[b], gsem.at[b])

    def gather_wait(b):
        pltpu.make_async_copy(table_hbm.at[idx_v.at[b]], rows_v.at[b],
                              gsem.at[b]).wait()

    def scatter_start(b):
        pltpu.async_copy(rows_v.at[b], out_hbm.at[oidx_v.at[b]], ssem.at[b])

    def scatter_wait(b):
        pltpu.make_async_copy(rows_v.at[b], out_hbm.at[oidx_v.at[b]],
                              ssem.at[b]).wait()

    def slot(g, b, has_next, has_pref, skip_free=False):
        b1, b2 = (b + 1) % _NBUF, (b + 2) % _NBUF
        gather_wait(b)
        pltpu.make_async_copy(oix_sl(g), oidx_v.at[b], osem.at[b]).wait()
        scatter_start(b)
        if has_pref:
            pltpu.async_copy(tok_sl(g + 2), idx_v.at[b2], isem.at[b2])
        if has_next:
            pltpu.make_async_copy(tok_sl(g + 1), idx_v.at[b1],
                                  isem.at[b1]).wait()
            if not skip_free:
                scatter_wait(b1)
            pltpu.async_copy(oix_sl(g + 1), oidx_v.at[b1], osem.at[b1])
            gather_start(b1)

    pltpu.async_copy(tok_sl(0), idx_v.at[0], isem.at[0])
    pltpu.async_copy(oix_sl(0), oidx_v.at[0], osem.at[0])
    pltpu.make_async_copy(tok_sl(0), idx_v.at[0], isem.at[0]).wait()
    gather_start(0)
    pltpu.async_copy(tok_sl(1), idx_v.at[1], isem.at[1])

    for g in range(_NBUF - 1):
        slot(g, g, True, True, skip_free=True)

    def body(p, carry):
        g0 = (_NBUF - 1) + p * _NBUF
        for boff in range(_NBUF):
            g = g0 + boff
            slot(g, (_NBUF - 1 + boff) % _NBUF, True, True)
        return carry

    n_mid = (num_chunks - _NBUF - 1) - (_NBUF - 1) + 1
    lax.fori_loop(0, n_mid // _NBUF, body, 0)

    for g in range(_NBUF - 1 + (n_mid // _NBUF) * _NBUF, num_chunks):
        slot(g, g % _NBUF, g + 1 < num_chunks, g + 2 < num_chunks)

    for b in range(_NBUF):
        scatter_wait(b)


def kernel(input_tokens, table):
    B, H = input_tokens.shape
    V, D = table.shape
    n = B * H
    flat = input_tokens.reshape(n).astype(jnp.int32)
    oidx = jnp.arange(n, dtype=jnp.int32) * 2
    assert n % (_NW * _CHUNK * _NBUF) == 0

    k = pl.kernel(
        _embed_body,
        out_type=jax.ShapeDtypeStruct((2 * n, D), table.dtype),
        mesh=plsc.VectorSubcoreMesh(core_axis_name="c", subcore_axis_name="s"),
        scratch_types=[
            pltpu.VMEM((_NBUF, _CHUNK), jnp.int32),
            pltpu.VMEM((_NBUF, _CHUNK), jnp.int32),
            pltpu.VMEM((_NBUF, _CHUNK, D), jnp.float32),
            pltpu.SemaphoreType.DMA((_NBUF,)),
            pltpu.SemaphoreType.DMA((_NBUF,)),
            pltpu.SemaphoreType.DMA((_NBUF,)),
            pltpu.SemaphoreType.DMA((_NBUF,)),
        ],
        compiler_params=pltpu.CompilerParams(use_tc_tiling_on_sc=False),
    )
    out2 = k(flat, oidx, table)
    out128 = out2.reshape(n, 2 * D)
    return out128[:, :D].reshape(B, H, D)
